# baseline (speedup 1.0000x reference)
import math

import jax
import jax.numpy as jnp
from jax import lax
from jax.experimental import pallas as pl
from jax.experimental.pallas import tpu as pltpu
from jax.experimental.pallas import tpu_sc as plsc

B = 4
N = 8192
D = 768
M = 64
SH = 64
NEG_INF = float("-inf")
CN = 512
CN2 = 1024

_NC = 2
_NS = 16
_L = 16


def _score_select_kernel(tf_ref, w1_ref, b1_ref, w2_ref, b2_ref,
                         sel_ref, valid_ref, sc_ref):
    i = pl.program_id(0)
    b1 = b1_ref[...]
    w2 = w2_ref[...]
    b2 = b2_ref[...]
    w1b = w1_ref[...].astype(jnp.bfloat16)
    tf_all = tf_ref[...].reshape(B * CN, D)
    h_all = lax.dot_general(tf_all.astype(jnp.bfloat16), w1b,
                            (((1,), (1,)), ((), ())),
                            preferred_element_type=jnp.float32) + b1
    h_all = jnp.maximum(h_all, 0.0)
    for b in range(B):
        s = jnp.sum(h_all[b * CN:(b + 1) * CN] * w2, axis=1) + b2[0, 0]
        sc_ref[b, pl.ds(i * CN, CN)] = s

    @pl.when(i == N // CN - 1)
    def _():
        _select_body(sc_ref, sel_ref, valid_ref)


def _select_body(s_ref, sel_ref, valid_ref):
    s = s_ref[...]
    meanv = jnp.sum(s, axis=1, keepdims=True) / N
    varv = jnp.sum((s - meanv) * (s - meanv), axis=1, keepdims=True) / N
    thr = meanv + 0.5 * jnp.sqrt(varv)
    selmask = s > thr
    count = jnp.sum(selmask.astype(jnp.int32), axis=1, keepdims=True)
    top = jnp.logical_or(count == 0, count > M)
    bits = lax.bitcast_convert_type(s, jnp.int32)
    key = jnp.where(bits < 0, bits ^ 0x7FFFFFFF, bits)

    def body(t, lohi):
        lo, hi = lohi
        mid = (lo & hi) + ((lo ^ hi) >> 1)
        cnt = jnp.sum((key > mid).astype(jnp.int32), axis=1, keepdims=True)
        adv = cnt >= M
        return jnp.where(adv, mid + 1, lo), jnp.where(adv, hi, mid)

    lo0 = jnp.full((B, 1), -2147483647 - 1, jnp.int32)
    hi0 = jnp.full((B, 1), 2147483647, jnp.int32)
    key64, _ = lax.fori_loop(0, 32, body, (lo0, hi0))
    vbits = jnp.where(key64 < 0, key64 ^ 0x7FFFFFFF, key64)
    val64 = lax.bitcast_convert_type(vbits, jnp.float32)
    g = jnp.sum((s > val64).astype(jnp.float32), axis=1, keepdims=True)
    quota_f = jnp.where(top, M - g, float(M))
    valid_ref[...] = jnp.where(count == 0, M, jnp.minimum(count, M))

    RR, CC = N // 128, 128
    f32 = jnp.float32
    lt = (lax.broadcasted_iota(jnp.int32, (CC, CC), 0)
          <= lax.broadcasted_iota(jnp.int32, (CC, CC), 1)).astype(f32)
    a64 = (lax.broadcasted_iota(jnp.int32, (RR, RR), 1)
           < lax.broadcasted_iota(jnp.int32, (RR, RR), 0)).astype(f32)
    kcol = lax.broadcasted_iota(jnp.int32, (M, 1), 0).astype(f32)
    klane = lax.broadcasted_iota(jnp.int32, (RR, M), 1).astype(f32)
    colidx = lax.broadcasted_iota(jnp.int32, (1, CC), 1).astype(f32)
    dn = (((1,), (0,)), ((), ()))
    dt = (((0,), (0,)), ((), ()))
    top_i = top.astype(jnp.int32)

    for b in range(B):
        s2 = s[b].reshape(RR, CC)
        topb = top_i[b, 0] != 0
        v64b = val64[b, 0]
        thrb = thr[b, 0]
        qb = quota_f[b, 0]
        gt = jnp.logical_and(topb, s2 > v64b)
        cand_f = jnp.where(topb, (s2 == v64b).astype(f32),
                           (s2 > thrb).astype(f32))
        cand = cand_f > 0.5
        pr = lax.dot_general(cand_f, lt, dn)
        rt = jnp.sum(cand_f, axis=1, keepdims=True)
        re = lax.dot_general(a64, rt, dn)
        take = jnp.logical_and(cand, (re + pr) <= qb)
        member = jnp.logical_or(gt, take)
        mem_f = member.astype(f32)
        pm = lax.dot_general(mem_f, lt, dn)
        rtm = jnp.sum(mem_f, axis=1, keepdims=True)
        rsm = lax.dot_general(a64, rtm, dn)
        e2 = jnp.logical_and(klane >= rsm, klane < rsm + rtm).astype(f32)
        r_col = lax.dot_general(e2, kcol, dt)
        rs_sel = lax.dot_general(e2, rsm, dt)
        l_col = kcol - rs_sel
        rowpm = lax.dot_general(e2, pm, dt)
        rowmem = lax.dot_general(e2, mem_f, dt)
        hit = jnp.logical_and(rowmem > 0.5, rowpm == l_col + 1.0)
        hit_f = hit.astype(f32)
        c_col = jnp.sum(hit_f * colidx, axis=1, keepdims=True)
        anyhit = jnp.sum(hit_f, axis=1, keepdims=True)
        sel_col = jnp.where(anyhit > 0.5, r_col * CC + c_col, float(N - 1))
        sel_ref[:, b:b + 1] = (sel_col + b * N).astype(jnp.int32)


_NW = _NC * _NS
_RPW = (B * M) // _NW


def _sc_gather_kernel(table_hbm, idx_hbm, out_hbm, idx_v, rows_v, sem):
    wid = lax.axis_index("s") * _NC + lax.axis_index("c")
    base = wid * _RPW
    pltpu.sync_copy(idx_hbm.at[pl.ds(base, _RPW)], idx_v)
    pltpu.async_copy(table_hbm.at[idx_v], rows_v, sem).wait()
    pltpu.sync_copy(rows_v, out_hbm.at[pl.ds(base, _RPW)])


def _nodes_body(raw_ref, t2n_ref, g1_ref, g2_ref, valid_ref, hg_ref):
    t2n = t2n_ref[...]
    g1 = g1_ref[...]
    g2 = g2_ref[...]
    ii = lax.broadcasted_iota(jnp.int32, (M, M), 0)
    jj = lax.broadcasted_iota(jnp.int32, (M, M), 1)
    for b in range(B):
        valid = valid_ref[b, 0]
        raw = raw_ref[b]
        rowmask = (lax.broadcasted_iota(jnp.int32, (M, 1), 0) < valid)
        h0 = lax.dot_general(raw, t2n, (((1,), (1,)), ((), ())))
        h0 = h0 * rowmask.astype(jnp.float32)
        nrm = jnp.maximum(jnp.sqrt(jnp.sum(h0 * h0, axis=1, keepdims=True)), 1e-6)
        hn = h0 / nrm
        sim = lax.dot_general(hn, hn, (((1,), (1,)), ((), ())))
        mm = jnp.logical_and(ii < valid, jj < valid)
        a = jnp.maximum(sim, 0.0) * mm.astype(jnp.float32)
        a = a + jnp.logical_and(ii == jj, jj < valid).astype(jnp.float32)
        rowsum = jnp.maximum(jnp.sum(a, axis=1, keepdims=True), 1e-6)
        an = a / rowsum
        x = jnp.dot(an, h0)
        x = jnp.maximum(lax.dot_general(x, g1, (((1,), (1,)), ((), ()))), 0.0)
        x = jnp.dot(an, x)
        hg = jnp.maximum(lax.dot_general(x, g2, (((1,), (1,)), ((), ()))), 0.0)
        hg_ref[b] = hg


def _nodes_inject_kernel(tf_ref, raw_ref, t2n_ref, n2t_ref, g1_ref, g2_ref,
                         valid_ref, out_ref, hg_ref):
    @pl.when(pl.program_id(0) == 0)
    def _():
        _nodes_body(raw_ref, t2n_ref, g1_ref, g2_ref, valid_ref, hg_ref)

    t2n = t2n_ref[...].astype(jnp.bfloat16)
    n2t = n2t_ref[...].astype(jnp.bfloat16)
    m_dyn = valid_ref[0, 0]
    for b in range(1, B):
        m_dyn = jnp.maximum(m_dyn, valid_ref[b, 0])
    scale = 1.0 / math.sqrt(float(M))
    tf_all = tf_ref[...].reshape(B * CN2, D)
    tp_all = lax.dot_general(tf_all.astype(jnp.bfloat16), t2n,
                             (((1,), (1,)), ((), ())),
                             preferred_element_type=jnp.float32)
    inj_list = []
    for b in range(B):
        tp = tp_all[b * CN2:(b + 1) * CN2]
        hg = hg_ref[b]
        lg = lax.dot_general(tp, hg, (((1,), (1,)), ((), ())))
        col = lax.broadcasted_iota(jnp.int32, lg.shape, 1)
        lg = jnp.where(col < m_dyn, lg, NEG_INF) * scale
        mx = jnp.max(lg, axis=1, keepdims=True)
        e = jnp.exp(lg - mx)
        attn = e / jnp.sum(e, axis=1, keepdims=True)
        inj_list.append(jnp.dot(attn, hg))
    inj_all = jnp.concatenate(inj_list, axis=0)
    back_all = lax.dot_general(inj_all.astype(jnp.bfloat16), n2t,
                               (((1,), (1,)), ((), ())),
                               preferred_element_type=jnp.float32)
    out_ref[...] = (tf_all + back_all).reshape(B, CN2, D)


def kernel(token_feats, score_w1, score_b1, score_w2, score_b2,
           t2n_W, n2t_W, gcn_W1, gcn_W2):
    f32 = jnp.float32
    i32 = jnp.int32

    sel, valid = pl.pallas_call(
        _score_select_kernel,
        grid=(N // CN,),
        in_specs=[
            pl.BlockSpec((B, CN, D), lambda i: (0, i, 0)),
            pl.BlockSpec((SH, D), lambda i: (0, 0)),
            pl.BlockSpec((1, SH), lambda i: (0, 0)),
            pl.BlockSpec((1, SH), lambda i: (0, 0)),
            pl.BlockSpec((1, 1), lambda i: (0, 0)),
        ],
        out_specs=(pl.BlockSpec((M, B), lambda i: (0, 0)),
                   pl.BlockSpec((B, 1), lambda i: (0, 0))),
        out_shape=(jax.ShapeDtypeStruct((M, B), i32),
                   jax.ShapeDtypeStruct((B, 1), i32)),
        scratch_shapes=[pltpu.VMEM((B, N), f32)],
    )(token_feats, score_w1, score_b1.reshape(1, SH), score_w2,
      score_b2.reshape(1, 1))
    sel_flat = jnp.transpose(sel).reshape(B * M)

    sc_gather = pl.kernel(
        _sc_gather_kernel,
        mesh=plsc.VectorSubcoreMesh(core_axis_name="c", subcore_axis_name="s"),
        out_type=jax.ShapeDtypeStruct((B * M, D), f32),
        scratch_types=[
            pltpu.VMEM((_RPW,), i32),
            pltpu.VMEM((_RPW, D), f32),
            pltpu.SemaphoreType.DMA,
        ],
    )
    nodes_raw = sc_gather(token_feats.reshape(B * N, D), sel_flat)
    nodes_raw = nodes_raw.reshape(B, M, D)

    out = pl.pallas_call(
        _nodes_inject_kernel,
        grid=(N // CN2,),
        in_specs=[
            pl.BlockSpec((B, CN2, D), lambda i: (0, i, 0)),
            pl.BlockSpec((B, M, D), lambda i: (0, 0, 0)),
            pl.BlockSpec((M, D), lambda i: (0, 0)),
            pl.BlockSpec((D, M), lambda i: (0, 0)),
            pl.BlockSpec((M, M), lambda i: (0, 0)),
            pl.BlockSpec((M, M), lambda i: (0, 0)),
            pl.BlockSpec(memory_space=pltpu.SMEM),
        ],
        out_specs=pl.BlockSpec((B, CN2, D), lambda i: (0, i, 0)),
        out_shape=jax.ShapeDtypeStruct((B, N, D), f32),
        scratch_shapes=[pltpu.VMEM((B, M, M), f32)],
    )(token_feats, nodes_raw, t2n_W, n2t_W, gcn_W1, gcn_W2, valid)

    return out


# speedup vs baseline: 1.1714x; 1.1714x over previous
import math

import jax
import jax.numpy as jnp
from jax import lax
from jax.experimental import pallas as pl
from jax.experimental.pallas import tpu as pltpu
from jax.experimental.pallas import tpu_sc as plsc

B = 4
N = 8192
D = 768
M = 64
SH = 64
NEG_INF = float("-inf")
CN = 512
CN2 = 1024

_NC = 2
_NS = 16
_L = 16


def _score_select_kernel(tf_ref, w1_ref, b1_ref, w2_ref, b2_ref,
                         sel_ref, valid_ref, sc_ref):
    i = pl.program_id(0)
    b1 = b1_ref[...]
    w2 = w2_ref[...]
    b2 = b2_ref[...]
    w1b = w1_ref[...].astype(jnp.bfloat16)
    for b in range(B):
        tfb = tf_ref[b]
        h = lax.dot_general(tfb.astype(jnp.bfloat16), w1b,
                            (((1,), (1,)), ((), ())),
                            preferred_element_type=jnp.float32) + b1
        h = jnp.maximum(h, 0.0)
        s = jnp.sum(h * w2, axis=1) + b2[0, 0]
        sc_ref[b, pl.ds(i * CN, CN)] = s

    @pl.when(i == N // CN - 1)
    def _():
        _select_body(sc_ref, sel_ref, valid_ref)


def _select_body(s_ref, sel_ref, valid_ref):
    s = s_ref[...]
    meanv = jnp.sum(s, axis=1, keepdims=True) / N
    varv = jnp.sum((s - meanv) * (s - meanv), axis=1, keepdims=True) / N
    thr = meanv + 0.5 * jnp.sqrt(varv)
    selmask = s > thr
    count = jnp.sum(selmask.astype(jnp.int32), axis=1, keepdims=True)
    top = jnp.logical_or(count == 0, count > M)
    bits = lax.bitcast_convert_type(s, jnp.int32)
    key = jnp.where(bits < 0, bits ^ 0x7FFFFFFF, bits)

    def body(t, lohi):
        lo, hi = lohi
        mid = (lo & hi) + ((lo ^ hi) >> 1)
        cnt = jnp.sum((key > mid).astype(jnp.int32), axis=1, keepdims=True)
        adv = cnt >= M
        return jnp.where(adv, mid + 1, lo), jnp.where(adv, hi, mid)

    lo0 = jnp.full((B, 1), -2147483647 - 1, jnp.int32)
    hi0 = jnp.full((B, 1), 2147483647, jnp.int32)
    key64, _ = lax.fori_loop(0, 32, body, (lo0, hi0))
    vbits = jnp.where(key64 < 0, key64 ^ 0x7FFFFFFF, key64)
    val64 = lax.bitcast_convert_type(vbits, jnp.float32)
    g = jnp.sum((s > val64).astype(jnp.float32), axis=1, keepdims=True)
    quota_f = jnp.where(top, M - g, float(M))
    valid_ref[...] = jnp.where(count == 0, M, jnp.minimum(count, M))

    RR, CC = N // 128, 128
    f32 = jnp.float32
    lt = (lax.broadcasted_iota(jnp.int32, (CC, CC), 0)
          <= lax.broadcasted_iota(jnp.int32, (CC, CC), 1)).astype(f32)
    a64 = (lax.broadcasted_iota(jnp.int32, (RR, RR), 1)
           < lax.broadcasted_iota(jnp.int32, (RR, RR), 0)).astype(f32)
    kcol = lax.broadcasted_iota(jnp.int32, (M, 1), 0).astype(f32)
    klane = lax.broadcasted_iota(jnp.int32, (RR, M), 1).astype(f32)
    colidx = lax.broadcasted_iota(jnp.int32, (1, CC), 1).astype(f32)
    dn = (((1,), (0,)), ((), ()))
    dt = (((0,), (0,)), ((), ()))
    top_i = top.astype(jnp.int32)

    for b in range(B):
        s2 = s[b].reshape(RR, CC)
        topb = top_i[b, 0] != 0
        v64b = val64[b, 0]
        thrb = thr[b, 0]
        qb = quota_f[b, 0]
        gt = jnp.logical_and(topb, s2 > v64b)
        cand_f = jnp.where(topb, (s2 == v64b).astype(f32),
                           (s2 > thrb).astype(f32))
        cand = cand_f > 0.5
        pr = lax.dot_general(cand_f, lt, dn)
        rt = jnp.sum(cand_f, axis=1, keepdims=True)
        re = lax.dot_general(a64, rt, dn)
        take = jnp.logical_and(cand, (re + pr) <= qb)
        member = jnp.logical_or(gt, take)
        mem_f = member.astype(f32)
        pm = lax.dot_general(mem_f, lt, dn)
        rtm = jnp.sum(mem_f, axis=1, keepdims=True)
        rsm = lax.dot_general(a64, rtm, dn)
        e2 = jnp.logical_and(klane >= rsm, klane < rsm + rtm).astype(f32)
        r_col = lax.dot_general(e2, kcol, dt)
        rs_sel = lax.dot_general(e2, rsm, dt)
        l_col = kcol - rs_sel
        rowpm = lax.dot_general(e2, pm, dt)
        rowmem = lax.dot_general(e2, mem_f, dt)
        hit = jnp.logical_and(rowmem > 0.5, rowpm == l_col + 1.0)
        hit_f = hit.astype(f32)
        c_col = jnp.sum(hit_f * colidx, axis=1, keepdims=True)
        anyhit = jnp.sum(hit_f, axis=1, keepdims=True)
        sel_col = jnp.where(anyhit > 0.5, r_col * CC + c_col, float(N - 1))
        sel_ref[:, b:b + 1] = (sel_col + b * N).astype(jnp.int32)


_NW = _NC * _NS
_RPW = (B * M) // _NW


def _sc_gather_kernel(table_hbm, idx_hbm, out_hbm, idx_v, rows_v, sem):
    wid = lax.axis_index("s") * _NC + lax.axis_index("c")
    base = wid * _RPW
    pltpu.sync_copy(idx_hbm.at[pl.ds(base, _RPW)], idx_v)
    pltpu.async_copy(table_hbm.at[idx_v], rows_v, sem).wait()
    pltpu.sync_copy(rows_v, out_hbm.at[pl.ds(base, _RPW)])


def _nodes_body(raw_ref, t2n_ref, g1_ref, g2_ref, valid_ref, hg_ref):
    t2n = t2n_ref[...]
    g1 = g1_ref[...]
    g2 = g2_ref[...]
    ii = lax.broadcasted_iota(jnp.int32, (M, M), 0)
    jj = lax.broadcasted_iota(jnp.int32, (M, M), 1)
    for b in range(B):
        valid = valid_ref[b, 0]
        raw = raw_ref[b]
        rowmask = (lax.broadcasted_iota(jnp.int32, (M, 1), 0) < valid)
        h0 = lax.dot_general(raw, t2n, (((1,), (1,)), ((), ())))
        h0 = h0 * rowmask.astype(jnp.float32)
        nrm = jnp.maximum(jnp.sqrt(jnp.sum(h0 * h0, axis=1, keepdims=True)), 1e-6)
        hn = h0 / nrm
        sim = lax.dot_general(hn, hn, (((1,), (1,)), ((), ())))
        mm = jnp.logical_and(ii < valid, jj < valid)
        a = jnp.maximum(sim, 0.0) * mm.astype(jnp.float32)
        a = a + jnp.logical_and(ii == jj, jj < valid).astype(jnp.float32)
        rowsum = jnp.maximum(jnp.sum(a, axis=1, keepdims=True), 1e-6)
        an = a / rowsum
        x = jnp.dot(an, h0)
        x = jnp.maximum(lax.dot_general(x, g1, (((1,), (1,)), ((), ()))), 0.0)
        x = jnp.dot(an, x)
        hg = jnp.maximum(lax.dot_general(x, g2, (((1,), (1,)), ((), ()))), 0.0)
        hg_ref[b] = hg


def _nodes_inject_kernel(tf_ref, raw_ref, t2n_ref, n2t_ref, g1_ref, g2_ref,
                         valid_ref, out_ref, hg_ref):
    @pl.when(pl.program_id(0) == 0)
    def _():
        _nodes_body(raw_ref, t2n_ref, g1_ref, g2_ref, valid_ref, hg_ref)

    t2n = t2n_ref[...].astype(jnp.bfloat16)
    n2t = n2t_ref[...].astype(jnp.bfloat16)
    m_dyn = valid_ref[0, 0]
    for b in range(1, B):
        m_dyn = jnp.maximum(m_dyn, valid_ref[b, 0])
    scale = 1.0 / math.sqrt(float(M))
    tf_all = tf_ref[...].reshape(B * CN2, D)
    tp_all = lax.dot_general(tf_all.astype(jnp.bfloat16), t2n,
                             (((1,), (1,)), ((), ())),
                             preferred_element_type=jnp.float32)
    inj_list = []
    for b in range(B):
        tp = tp_all[b * CN2:(b + 1) * CN2]
        hg = hg_ref[b]
        lg = lax.dot_general(tp, hg, (((1,), (1,)), ((), ())))
        col = lax.broadcasted_iota(jnp.int32, lg.shape, 1)
        lg = jnp.where(col < m_dyn, lg, NEG_INF) * scale
        mx = jnp.max(lg, axis=1, keepdims=True)
        e = jnp.exp(lg - mx)
        attn = e / jnp.sum(e, axis=1, keepdims=True)
        inj_list.append(jnp.dot(attn, hg))
    inj_all = jnp.concatenate(inj_list, axis=0)
    back_all = lax.dot_general(inj_all.astype(jnp.bfloat16), n2t,
                               (((1,), (1,)), ((), ())),
                               preferred_element_type=jnp.float32)
    out_ref[...] = (tf_all + back_all).reshape(B, CN2, D)


def kernel(token_feats, score_w1, score_b1, score_w2, score_b2,
           t2n_W, n2t_W, gcn_W1, gcn_W2):
    f32 = jnp.float32
    i32 = jnp.int32

    sel, valid = pl.pallas_call(
        _score_select_kernel,
        grid=(N // CN,),
        in_specs=[
            pl.BlockSpec((B, CN, D), lambda i: (0, i, 0)),
            pl.BlockSpec((SH, D), lambda i: (0, 0)),
            pl.BlockSpec((1, SH), lambda i: (0, 0)),
            pl.BlockSpec((1, SH), lambda i: (0, 0)),
            pl.BlockSpec((1, 1), lambda i: (0, 0)),
        ],
        out_specs=(pl.BlockSpec((M, B), lambda i: (0, 0)),
                   pl.BlockSpec((B, 1), lambda i: (0, 0))),
        out_shape=(jax.ShapeDtypeStruct((M, B), i32),
                   jax.ShapeDtypeStruct((B, 1), i32)),
        scratch_shapes=[pltpu.VMEM((B, N), f32)],
    )(token_feats, score_w1, score_b1.reshape(1, SH), score_w2,
      score_b2.reshape(1, 1))
    sel_flat = jnp.transpose(sel).reshape(B * M)

    sc_gather = pl.kernel(
        _sc_gather_kernel,
        mesh=plsc.VectorSubcoreMesh(core_axis_name="c", subcore_axis_name="s"),
        out_type=jax.ShapeDtypeStruct((B * M, D), f32),
        scratch_types=[
            pltpu.VMEM((_RPW,), i32),
            pltpu.VMEM((_RPW, D), f32),
            pltpu.SemaphoreType.DMA,
        ],
    )
    nodes_raw = sc_gather(token_feats.reshape(B * N, D), sel_flat)
    nodes_raw = nodes_raw.reshape(B, M, D)

    out = pl.pallas_call(
        _nodes_inject_kernel,
        grid=(N // CN2,),
        in_specs=[
            pl.BlockSpec((B, CN2, D), lambda i: (0, i, 0)),
            pl.BlockSpec((B, M, D), lambda i: (0, 0, 0)),
            pl.BlockSpec((M, D), lambda i: (0, 0)),
            pl.BlockSpec((D, M), lambda i: (0, 0)),
            pl.BlockSpec((M, M), lambda i: (0, 0)),
            pl.BlockSpec((M, M), lambda i: (0, 0)),
            pl.BlockSpec(memory_space=pltpu.SMEM),
        ],
        out_specs=pl.BlockSpec((B, CN2, D), lambda i: (0, i, 0)),
        out_shape=jax.ShapeDtypeStruct((B, N, D), f32),
        scratch_shapes=[pltpu.VMEM((B, M, M), f32)],
    )(token_feats, nodes_raw, t2n_W, n2t_W, gcn_W1, gcn_W2, valid)

    return out
